# baseline (speedup 1.0000x reference)
import functools

import jax
import jax.numpy as jnp
from jax import lax
from jax.experimental import pallas as pl
from jax.experimental.pallas import tpu as pltpu
from jax.experimental.pallas import tpu_sc as plsc

BATCH = 16384
EMBED = 64
NGENRE = 26
OUTD = 2 * EMBED
LANES = 16
GPB = 2


@functools.cache
def _make_fused():
    info = plsc.get_sparse_core_info()
    nc, ns = info.num_cores, info.num_subcores
    nw = nc * ns
    bpw = BATCH // nw
    nblk = bpw // (GPB * LANES)
    mesh = plsc.VectorSubcoreMesh(core_axis_name="c", subcore_axis_name="s")

    @functools.partial(
        pl.kernel,
        mesh=mesh,
        out_type=jax.ShapeDtypeStruct((BATCH * OUTD,), jnp.float32),
        scratch_types=[
            pltpu.VMEM((bpw,), jnp.int32),
            pltpu.VMEM((bpw, EMBED), jnp.float32),
            pltpu.VMEM((bpw * NGENRE,), jnp.int32),
            pltpu.VMEM((EMBED * NGENRE,), jnp.float32),
            pltpu.VMEM((bpw * OUTD,), jnp.float32),
            pltpu.SemaphoreType.DMA,
        ],
        compiler_params=pltpu.CompilerParams(
            use_tc_tiling_on_sc=False, needs_layout_passes=False),
    )
    def fused(table_hbm, idx_hbm, g_hbm, w_hbm, out_hbm,
              idx_v, rows_v, g_v, w_v, comb_v, sem):
        wid = lax.axis_index("s") * nc + lax.axis_index("c")
        base = wid * bpw
        pltpu.sync_copy(idx_hbm.at[pl.ds(base, bpw)], idx_v)
        gather = pltpu.async_copy(table_hbm.at[idx_v], rows_v, sem)
        pltpu.sync_copy(g_hbm.at[pl.ds(base * NGENRE, bpw * NGENRE)], g_v)
        pltpu.sync_copy(w_hbm, w_v)

        lane = lax.iota(jnp.int32, 16)
        lane26 = lane * NGENRE
        lane128 = lane * OUTD
        one = jnp.float32(1.0)

        def block(b, carry):
            r0 = b * (GPB * LANES)
            gcols = []
            invs = []
            svecs = []
            for g in range(GPB):
                rg = r0 + g * LANES
                gbase = lane26 + rg * NGENRE
                cols = [plsc.load_gather(g_v, [gbase + j]).astype(jnp.float32)
                        for j in range(NGENRE)]
                cnt = cols[0]
                for j in range(1, NGENRE):
                    cnt = cnt + cols[j]
                gcols.append(cols)
                invs.append(one / cnt)
                svecs.append(lane128 + (rg * OUTD + EMBED))

            def dbody(d, carry2):
                wbase = jnp.broadcast_to(d * NGENRE, (16,)).astype(jnp.int32)
                w0 = plsc.load_gather(w_v, [wbase])
                accs = [gcols[g][0] * w0 for g in range(GPB)]
                for j in range(1, NGENRE):
                    w = plsc.load_gather(w_v, [wbase + j])
                    for g in range(GPB):
                        accs[g] = accs[g] + gcols[g][j] * w
                for g in range(GPB):
                    plsc.store_scatter(comb_v, [svecs[g] + d],
                                       accs[g] * invs[g])
                return carry2

            return lax.fori_loop(0, EMBED, dbody, carry)

        lax.fori_loop(0, nblk, block, 0)
        gather.wait()

        def yrow(r, carry):
            o = r * OUTD
            for c in range(EMBED // LANES):
                comb_v[pl.ds(o + c * LANES, LANES)] = rows_v[r, pl.ds(c * LANES, LANES)]
            return carry

        lax.fori_loop(0, bpw, yrow, 0)
        pltpu.sync_copy(comb_v, out_hbm.at[pl.ds(base * OUTD, bpw * OUTD)])

    return fused


def kernel(year_idx, genre_idx, embedding_year, W_genre):
    idx = year_idx.astype(jnp.int32)
    g_flat = genre_idx.reshape(-1)
    out = _make_fused()(embedding_year, idx, g_flat, W_genre.reshape(-1))
    return out.reshape(BATCH, OUTD)
